# baseline (device time: 1412903 ns/iter reference)
import jax
import jax.numpy as jnp
from jax import lax
from jax.experimental import pallas as pl
from jax.experimental.pallas import tpu as pltpu

N_DEV = 16


def kernel(x, w_mat):
    m_total, k_per = x.shape
    k_per2, n = w_mat.shape
    assert k_per == k_per2
    m_per = m_total // N_DEV

    def body(x_ref, w_ref, out_ref, send_buf, recv_buf, send_sem, recv_sems,
             credit_sem):
        my_pos = lax.axis_index("i")
        right = lax.rem(my_pos + 1, N_DEV)
        left = lax.rem(my_pos + N_DEV - 1, N_DEV)

        def partial_chunk(c):
            rows = x_ref[pl.ds(c * m_per, m_per), :]
            return jnp.dot(rows, w_ref[...], preferred_element_type=jnp.float32)

        barrier = pltpu.get_barrier_semaphore()
        for nbr in (left, right):
            pl.semaphore_signal(barrier, inc=1, device_id=(nbr,),
                                device_id_type=pl.DeviceIdType.MESH)
        pl.semaphore_wait(barrier, 2)

        send_buf[...] = partial_chunk(left)

        for h in range(N_DEV - 1):
            slot = h % 2
            if h >= 2:
                pl.semaphore_wait(credit_sem, 1)
            rdma = pltpu.make_async_remote_copy(
                src_ref=send_buf,
                dst_ref=recv_buf.at[slot],
                send_sem=send_sem,
                recv_sem=recv_sems.at[slot],
                device_id=(right,),
                device_id_type=pl.DeviceIdType.MESH,
            )
            rdma.start()
            c = lax.rem(my_pos + 2 * N_DEV - 2 - h, N_DEV)
            p = partial_chunk(c)
            rdma.wait_recv()
            acc = recv_buf[slot] + p
            rdma.wait_send()
            if h < N_DEV - 2:
                send_buf[...] = acc
            else:
                out_ref[...] = acc * jax.nn.sigmoid(acc)
            if h <= N_DEV - 4:
                pl.semaphore_signal(credit_sem, inc=1, device_id=(left,),
                                    device_id_type=pl.DeviceIdType.MESH)

    return pl.pallas_call(
        body,
        out_shape=jax.ShapeDtypeStruct((m_per, n), jnp.float32),
        in_specs=[
            pl.BlockSpec(memory_space=pltpu.VMEM),
            pl.BlockSpec(memory_space=pltpu.VMEM),
        ],
        out_specs=pl.BlockSpec(memory_space=pltpu.VMEM),
        scratch_shapes=[
            pltpu.VMEM((m_per, n), jnp.float32),
            pltpu.VMEM((2, m_per, n), jnp.float32),
            pltpu.SemaphoreType.DMA,
            pltpu.SemaphoreType.DMA((2,)),
            pltpu.SemaphoreType.REGULAR,
        ],
        compiler_params=pltpu.CompilerParams(collective_id=0),
    )(x, w_mat)


# device time: 755555 ns/iter; 1.8700x vs baseline; 1.8700x over previous
import jax
import jax.numpy as jnp
from jax import lax
from jax.experimental import pallas as pl
from jax.experimental.pallas import tpu as pltpu

N_DEV = 16


def kernel(x, w_mat):
    m_total, k_per = x.shape
    k_per2, n = w_mat.shape
    assert k_per == k_per2
    m_per = m_total // N_DEV
    nh = n // 2

    def body(x_ref, w_ref, out_ref, send_a, send_b, recv_a, recv_b,
             send_sem_a, send_sem_b, recv_sems_a, recv_sems_b,
             credit_a, credit_b):
        my_pos = lax.axis_index("i")
        right = lax.rem(my_pos + 1, N_DEV)
        left = lax.rem(my_pos + N_DEV - 1, N_DEV)

        def partial_a(c):
            rows = x_ref[pl.ds(c * m_per, m_per), :]
            return jnp.dot(rows, w_ref[:, 0:nh],
                           preferred_element_type=jnp.float32)

        def partial_b(c):
            rows = x_ref[pl.ds(c * m_per, m_per), :]
            return jnp.dot(rows, w_ref[:, nh:n],
                           preferred_element_type=jnp.float32)

        barrier = pltpu.get_barrier_semaphore()
        for nbr in (left, right):
            pl.semaphore_signal(barrier, inc=1, device_id=(nbr,),
                                device_id_type=pl.DeviceIdType.MESH)
        pl.semaphore_wait(barrier, 2)

        send_a[...] = partial_a(left)
        send_b[...] = partial_b(right)

        for h in range(N_DEV - 1):
            slot = h % 2
            if h >= 2:
                pl.semaphore_wait(credit_a, 1)
                pl.semaphore_wait(credit_b, 1)
            rdma_a = pltpu.make_async_remote_copy(
                src_ref=send_a,
                dst_ref=recv_a.at[slot],
                send_sem=send_sem_a,
                recv_sem=recv_sems_a.at[slot],
                device_id=(right,),
                device_id_type=pl.DeviceIdType.MESH,
            )
            rdma_b = pltpu.make_async_remote_copy(
                src_ref=send_b,
                dst_ref=recv_b.at[slot],
                send_sem=send_sem_b,
                recv_sem=recv_sems_b.at[slot],
                device_id=(left,),
                device_id_type=pl.DeviceIdType.MESH,
            )
            rdma_a.start()
            rdma_b.start()
            c_a = lax.rem(my_pos + 2 * N_DEV - 2 - h, N_DEV)
            c_b = lax.rem(my_pos + 2 + h, N_DEV)
            p_a = partial_a(c_a)
            p_b = partial_b(c_b)
            rdma_a.wait_recv()
            acc_a = recv_a[slot] + p_a
            rdma_b.wait_recv()
            acc_b = recv_b[slot] + p_b
            rdma_a.wait_send()
            rdma_b.wait_send()
            if h < N_DEV - 2:
                send_a[...] = acc_a
                send_b[...] = acc_b
            else:
                out_ref[:, 0:nh] = acc_a * jax.nn.sigmoid(acc_a)
                out_ref[:, nh:n] = acc_b * jax.nn.sigmoid(acc_b)
            if h <= N_DEV - 4:
                pl.semaphore_signal(credit_a, inc=1, device_id=(left,),
                                    device_id_type=pl.DeviceIdType.MESH)
                pl.semaphore_signal(credit_b, inc=1, device_id=(right,),
                                    device_id_type=pl.DeviceIdType.MESH)

    return pl.pallas_call(
        body,
        out_shape=jax.ShapeDtypeStruct((m_per, n), jnp.float32),
        in_specs=[
            pl.BlockSpec(memory_space=pltpu.VMEM),
            pl.BlockSpec(memory_space=pltpu.VMEM),
        ],
        out_specs=pl.BlockSpec(memory_space=pltpu.VMEM),
        scratch_shapes=[
            pltpu.VMEM((m_per, nh), jnp.float32),
            pltpu.VMEM((m_per, nh), jnp.float32),
            pltpu.VMEM((2, m_per, nh), jnp.float32),
            pltpu.VMEM((2, m_per, nh), jnp.float32),
            pltpu.SemaphoreType.DMA,
            pltpu.SemaphoreType.DMA,
            pltpu.SemaphoreType.DMA((2,)),
            pltpu.SemaphoreType.DMA((2,)),
            pltpu.SemaphoreType.REGULAR,
            pltpu.SemaphoreType.REGULAR,
        ],
        compiler_params=pltpu.CompilerParams(collective_id=0),
    )(x, w_mat)


# device time: 734618 ns/iter; 1.9233x vs baseline; 1.0285x over previous
import jax
import jax.numpy as jnp
from jax import lax
from jax.experimental import pallas as pl
from jax.experimental.pallas import tpu as pltpu

N_DEV = 16


def kernel(x, w_mat):
    m_total, k_per = x.shape
    k_per2, n = w_mat.shape
    assert k_per == k_per2
    m_per = m_total // N_DEV
    nh = n // 2

    def body(x_ref, w_ref, out_ref, send_a, send_b, recv_a, recv_b,
             send_sems_a, send_sems_b, recv_sems_a, recv_sems_b,
             credit_a, credit_b):
        my_pos = lax.axis_index("i")
        right = lax.rem(my_pos + 1, N_DEV)
        left = lax.rem(my_pos + N_DEV - 1, N_DEV)

        def partial_a(c):
            rows = x_ref[pl.ds(c * m_per, m_per), :]
            return jnp.dot(rows, w_ref[:, 0:nh],
                           preferred_element_type=jnp.float32)

        def partial_b(c):
            rows = x_ref[pl.ds(c * m_per, m_per), :]
            return jnp.dot(rows, w_ref[:, nh:n],
                           preferred_element_type=jnp.float32)

        def make_a(slot):
            return pltpu.make_async_remote_copy(
                src_ref=send_a.at[slot],
                dst_ref=recv_a.at[slot],
                send_sem=send_sems_a.at[slot],
                recv_sem=recv_sems_a.at[slot],
                device_id=(right,),
                device_id_type=pl.DeviceIdType.MESH,
            )

        def make_b(slot):
            return pltpu.make_async_remote_copy(
                src_ref=send_b.at[slot],
                dst_ref=recv_b.at[slot],
                send_sem=send_sems_b.at[slot],
                recv_sem=recv_sems_b.at[slot],
                device_id=(left,),
                device_id_type=pl.DeviceIdType.MESH,
            )

        barrier = pltpu.get_barrier_semaphore()
        for nbr in (left, right):
            pl.semaphore_signal(barrier, inc=1, device_id=(nbr,),
                                device_id_type=pl.DeviceIdType.MESH)
        pl.semaphore_wait(barrier, 2)

        send_a[0] = partial_a(left)
        cur_a = make_a(0)
        cur_a.start()
        send_b[0] = partial_b(right)
        cur_b = make_b(0)
        cur_b.start()

        prev_a = prev_b = None
        for h in range(N_DEV - 1):
            slot = h % 2
            nslot = (h + 1) % 2
            last = h == N_DEV - 2
            c_a = lax.rem(my_pos + 2 * N_DEV - 2 - h, N_DEV)
            c_b = lax.rem(my_pos + 2 + h, N_DEV)

            p_a = partial_a(c_a)
            cur_a.wait_recv()
            if last:
                acc = recv_a[slot] + p_a
                out_ref[:, 0:nh] = acc * jax.nn.sigmoid(acc)
            else:
                if prev_a is not None:
                    prev_a.wait_send()
                send_a[nslot] = recv_a[slot] + p_a
            if h <= N_DEV - 4:
                pl.semaphore_signal(credit_a, inc=1, device_id=(left,),
                                    device_id_type=pl.DeviceIdType.MESH)
            if not last:
                if h + 1 >= 2:
                    pl.semaphore_wait(credit_a, 1)
                nxt_a = make_a(nslot)
                nxt_a.start()
                prev_a, cur_a = cur_a, nxt_a

            p_b = partial_b(c_b)
            cur_b.wait_recv()
            if last:
                acc = recv_b[slot] + p_b
                out_ref[:, nh:n] = acc * jax.nn.sigmoid(acc)
            else:
                if prev_b is not None:
                    prev_b.wait_send()
                send_b[nslot] = recv_b[slot] + p_b
            if h <= N_DEV - 4:
                pl.semaphore_signal(credit_b, inc=1, device_id=(right,),
                                    device_id_type=pl.DeviceIdType.MESH)
            if not last:
                if h + 1 >= 2:
                    pl.semaphore_wait(credit_b, 1)
                nxt_b = make_b(nslot)
                nxt_b.start()
                prev_b, cur_b = cur_b, nxt_b

        prev_a.wait_send()
        cur_a.wait_send()
        prev_b.wait_send()
        cur_b.wait_send()

    return pl.pallas_call(
        body,
        out_shape=jax.ShapeDtypeStruct((m_per, n), jnp.float32),
        in_specs=[
            pl.BlockSpec(memory_space=pltpu.VMEM),
            pl.BlockSpec(memory_space=pltpu.VMEM),
        ],
        out_specs=pl.BlockSpec(memory_space=pltpu.VMEM),
        scratch_shapes=[
            pltpu.VMEM((2, m_per, nh), jnp.float32),
            pltpu.VMEM((2, m_per, nh), jnp.float32),
            pltpu.VMEM((2, m_per, nh), jnp.float32),
            pltpu.VMEM((2, m_per, nh), jnp.float32),
            pltpu.SemaphoreType.DMA((2,)),
            pltpu.SemaphoreType.DMA((2,)),
            pltpu.SemaphoreType.DMA((2,)),
            pltpu.SemaphoreType.DMA((2,)),
            pltpu.SemaphoreType.REGULAR,
            pltpu.SemaphoreType.REGULAR,
        ],
        compiler_params=pltpu.CompilerParams(
            collective_id=0,
            vmem_limit_bytes=96 * 1024 * 1024,
        ),
    )(x, w_mat)


# device time: 698282 ns/iter; 2.0234x vs baseline; 1.0520x over previous
import jax
import jax.numpy as jnp
from jax import lax
from jax.experimental import pallas as pl
from jax.experimental.pallas import tpu as pltpu

N_DEV = 16
SUBS = 2


def kernel(x, w_mat):
    m_total, k_per = x.shape
    k_per2, n = w_mat.shape
    assert k_per == k_per2
    m_per = m_total // N_DEV
    nh = n // 2
    nq = nh // SUBS

    def body(x_ref, w_ref, out_ref, send_a, send_b, recv_a, recv_b,
             send_sems_a, send_sems_b, recv_sems_a, recv_sems_b,
             credit_a, credit_b):
        my_pos = lax.axis_index("i")
        right = lax.rem(my_pos + 1, N_DEV)
        left = lax.rem(my_pos + N_DEV - 1, N_DEV)

        rings = {
            "a": dict(send=send_a, recv=recv_a, ssem=send_sems_a,
                      rsem=recv_sems_a, credit=credit_a, dst=right,
                      credit_dst=left, col0=0),
            "b": dict(send=send_b, recv=recv_b, ssem=send_sems_b,
                      rsem=recv_sems_b, credit=credit_b, dst=left,
                      credit_dst=right, col0=nh),
        }
        ORDER = [("a", 0), ("b", 0), ("a", 1), ("b", 1)]

        def make(r, slot, sub):
            rc = rings[r]
            return pltpu.make_async_remote_copy(
                src_ref=rc["send"].at[slot, :, pl.ds(sub * nq, nq)],
                dst_ref=rc["recv"].at[slot, :, pl.ds(sub * nq, nq)],
                send_sem=rc["ssem"].at[slot, sub],
                recv_sem=rc["rsem"].at[slot, sub],
                device_id=(rc["dst"],),
                device_id_type=pl.DeviceIdType.MESH,
            )

        def chunk(r, h):
            if r == "a":
                return lax.rem(my_pos + 2 * N_DEV - 2 - h, N_DEV)
            return lax.rem(my_pos + 2 + h, N_DEV)

        def partial(r, c, sub):
            col = rings[r]["col0"] + sub * nq
            rows = x_ref[pl.ds(c * m_per, m_per), :]
            return jnp.dot(rows, w_ref[:, col:col + nq],
                           preferred_element_type=jnp.float32)

        barrier = pltpu.get_barrier_semaphore()
        for nbr in (left, right):
            pl.semaphore_signal(barrier, inc=1, device_id=(nbr,),
                                device_id_type=pl.DeviceIdType.MESH)
        pl.semaphore_wait(barrier, 2)

        seed_c = {"a": left, "b": right}
        cur = {}
        prev = {key: None for key in ORDER}
        for r, sub in ORDER:
            rings[r]["send"][0, :, sub * nq:(sub + 1) * nq] = (
                partial(r, seed_c[r], sub))
            d = make(r, 0, sub)
            d.start()
            cur[(r, sub)] = d

        for h in range(N_DEV - 1):
            slot = h % 2
            nslot = (h + 1) % 2
            last = h == N_DEV - 2
            for r, sub in ORDER:
                rc = rings[r]
                col = sub * nq
                p = partial(r, chunk(r, h), sub)
                cur[(r, sub)].wait_recv()
                if last:
                    acc = rc["recv"][slot, :, col:col + nq] + p
                    out_ref[:, rc["col0"] + col:rc["col0"] + col + nq] = (
                        acc * jax.nn.sigmoid(acc))
                else:
                    if prev[(r, sub)] is not None:
                        prev[(r, sub)].wait_send()
                    rc["send"][nslot, :, col:col + nq] = (
                        rc["recv"][slot, :, col:col + nq] + p)
                if sub == SUBS - 1 and h <= N_DEV - 4:
                    pl.semaphore_signal(rc["credit"], inc=1,
                                        device_id=(rc["credit_dst"],),
                                        device_id_type=pl.DeviceIdType.MESH)
                if not last:
                    if sub == 0 and h + 1 >= 2:
                        pl.semaphore_wait(rc["credit"], 1)
                    d = make(r, nslot, sub)
                    d.start()
                    prev[(r, sub)], cur[(r, sub)] = cur[(r, sub)], d

        for key in ORDER:
            prev[key].wait_send()
            cur[key].wait_send()

    return pl.pallas_call(
        body,
        out_shape=jax.ShapeDtypeStruct((m_per, n), jnp.float32),
        in_specs=[
            pl.BlockSpec(memory_space=pltpu.VMEM),
            pl.BlockSpec(memory_space=pltpu.VMEM),
        ],
        out_specs=pl.BlockSpec(memory_space=pltpu.VMEM),
        scratch_shapes=[
            pltpu.VMEM((2, m_per, nh), jnp.float32),
            pltpu.VMEM((2, m_per, nh), jnp.float32),
            pltpu.VMEM((2, m_per, nh), jnp.float32),
            pltpu.VMEM((2, m_per, nh), jnp.float32),
            pltpu.SemaphoreType.DMA((2, SUBS)),
            pltpu.SemaphoreType.DMA((2, SUBS)),
            pltpu.SemaphoreType.DMA((2, SUBS)),
            pltpu.SemaphoreType.DMA((2, SUBS)),
            pltpu.SemaphoreType.REGULAR,
            pltpu.SemaphoreType.REGULAR,
        ],
        compiler_params=pltpu.CompilerParams(
            collective_id=0,
            vmem_limit_bytes=96 * 1024 * 1024,
        ),
    )(x, w_mat)


# device time: 697339 ns/iter; 2.0261x vs baseline; 1.0014x over previous
import jax
import jax.numpy as jnp
from jax import lax
from jax.experimental import pallas as pl
from jax.experimental.pallas import tpu as pltpu

N_DEV = 16
SUBS = 4


def kernel(x, w_mat):
    m_total, k_per = x.shape
    k_per2, n = w_mat.shape
    assert k_per == k_per2
    m_per = m_total // N_DEV
    nh = n // 2
    nq = nh // SUBS

    def body(x_ref, w_ref, out_ref, send_a, send_b, recv_a, recv_b,
             send_sems_a, send_sems_b, recv_sems_a, recv_sems_b,
             credit_a, credit_b):
        my_pos = lax.axis_index("i")
        right = lax.rem(my_pos + 1, N_DEV)
        left = lax.rem(my_pos + N_DEV - 1, N_DEV)

        rings = {
            "a": dict(send=send_a, recv=recv_a, ssem=send_sems_a,
                      rsem=recv_sems_a, credit=credit_a, dst=right,
                      credit_dst=left, col0=0),
            "b": dict(send=send_b, recv=recv_b, ssem=send_sems_b,
                      rsem=recv_sems_b, credit=credit_b, dst=left,
                      credit_dst=right, col0=nh),
        }
        ORDER = [(r, s) for s in range(SUBS) for r in ("a", "b")]

        def make(r, slot, sub):
            rc = rings[r]
            return pltpu.make_async_remote_copy(
                src_ref=rc["send"].at[slot, :, pl.ds(sub * nq, nq)],
                dst_ref=rc["recv"].at[slot, :, pl.ds(sub * nq, nq)],
                send_sem=rc["ssem"].at[slot, sub],
                recv_sem=rc["rsem"].at[slot, sub],
                device_id=(rc["dst"],),
                device_id_type=pl.DeviceIdType.MESH,
            )

        def chunk(r, h):
            if r == "a":
                return lax.rem(my_pos + 2 * N_DEV - 2 - h, N_DEV)
            return lax.rem(my_pos + 2 + h, N_DEV)

        def partial(r, c, sub):
            col = rings[r]["col0"] + sub * nq
            rows = x_ref[pl.ds(c * m_per, m_per), :]
            return jnp.dot(rows, w_ref[:, col:col + nq],
                           preferred_element_type=jnp.float32)

        barrier = pltpu.get_barrier_semaphore()
        for nbr in (left, right):
            pl.semaphore_signal(barrier, inc=1, device_id=(nbr,),
                                device_id_type=pl.DeviceIdType.MESH)
        pl.semaphore_wait(barrier, 2)

        seed_c = {"a": left, "b": right}
        cur = {}
        prev = {key: None for key in ORDER}
        for r, sub in ORDER:
            rings[r]["send"][0, :, sub * nq:(sub + 1) * nq] = (
                partial(r, seed_c[r], sub))
            d = make(r, 0, sub)
            d.start()
            cur[(r, sub)] = d

        for h in range(N_DEV - 1):
            slot = h % 2
            nslot = (h + 1) % 2
            last = h == N_DEV - 2
            for r, sub in ORDER:
                rc = rings[r]
                col = sub * nq
                p = partial(r, chunk(r, h), sub)
                cur[(r, sub)].wait_recv()
                if last:
                    acc = rc["recv"][slot, :, col:col + nq] + p
                    out_ref[:, rc["col0"] + col:rc["col0"] + col + nq] = (
                        acc * jax.nn.sigmoid(acc))
                else:
                    if prev[(r, sub)] is not None:
                        prev[(r, sub)].wait_send()
                    rc["send"][nslot, :, col:col + nq] = (
                        rc["recv"][slot, :, col:col + nq] + p)
                if sub == SUBS - 1 and h <= N_DEV - 4:
                    pl.semaphore_signal(rc["credit"], inc=1,
                                        device_id=(rc["credit_dst"],),
                                        device_id_type=pl.DeviceIdType.MESH)
                if not last:
                    if sub == 0 and h + 1 >= 2:
                        pl.semaphore_wait(rc["credit"], 1)
                    d = make(r, nslot, sub)
                    d.start()
                    prev[(r, sub)], cur[(r, sub)] = cur[(r, sub)], d

        for key in ORDER:
            prev[key].wait_send()
            cur[key].wait_send()

    return pl.pallas_call(
        body,
        out_shape=jax.ShapeDtypeStruct((m_per, n), jnp.float32),
        in_specs=[
            pl.BlockSpec(memory_space=pltpu.VMEM),
            pl.BlockSpec(memory_space=pltpu.VMEM),
        ],
        out_specs=pl.BlockSpec(memory_space=pltpu.VMEM),
        scratch_shapes=[
            pltpu.VMEM((2, m_per, nh), jnp.float32),
            pltpu.VMEM((2, m_per, nh), jnp.float32),
            pltpu.VMEM((2, m_per, nh), jnp.float32),
            pltpu.VMEM((2, m_per, nh), jnp.float32),
            pltpu.SemaphoreType.DMA((2, SUBS)),
            pltpu.SemaphoreType.DMA((2, SUBS)),
            pltpu.SemaphoreType.DMA((2, SUBS)),
            pltpu.SemaphoreType.DMA((2, SUBS)),
            pltpu.SemaphoreType.REGULAR,
            pltpu.SemaphoreType.REGULAR,
        ],
        compiler_params=pltpu.CompilerParams(
            collective_id=0,
            vmem_limit_bytes=96 * 1024 * 1024,
        ),
    )(x, w_mat)
